# Initial kernel scaffold; baseline (speedup 1.0000x reference)
import jax
import jax.numpy as jnp
from jax.experimental import pallas as pl


def kernel(x, edge_index, W1, b1, W2, b2):
    raise NotImplementedError("write your pallas kernel here")


# speedup vs baseline: 39.0275x; 39.0275x over previous
import functools

import jax
import jax.numpy as jnp
from jax import lax
from jax.experimental import pallas as pl
from jax.experimental.pallas import tpu as pltpu
from jax.experimental.pallas import tpu_sc as plsc

N = 10000
E = 320000
DF = 128
DH = 16
NCLS = 16

NCORE = 2
NSUB = 16
NTILE = NCORE * NSUB

N_PAD = 10240
ROWS_PER_TILE = N_PAD // NSUB
BATCH = 128
E_PAD = 327680
NCHUNK = E_PAD // (NTILE * BATCH)

NB = 1000
GRID = N // NB

_mesh = plsc.VectorSubcoreMesh(core_axis_name="c", subcore_axis_name="s")
_sc_params = pltpu.CompilerParams(use_tc_tiling_on_sc=False)


@functools.partial(
    pl.kernel,
    out_type=jax.ShapeDtypeStruct((NCORE, N_PAD), jnp.float32),
    mesh=_mesh,
    scratch_types=[
        pltpu.VMEM_SHARED((N_PAD,), jnp.float32),
        pltpu.VMEM((NCHUNK, BATCH), jnp.int32),
        pltpu.VMEM((BATCH,), jnp.float32),
        pltpu.VMEM((ROWS_PER_TILE,), jnp.float32),
        pltpu.SemaphoreType.DMA,
    ],
    compiler_params=_sc_params,
)
def _deg_kernel(dst_hbm, out_hbm, acc, didx, ones_v, stage, sem):
    cid = lax.axis_index("c")
    sid = lax.axis_index("s")
    tile = cid * NSUB + sid

    def _fill(i, _):
        ones_v[pl.ds(i * 16, 16)] = jnp.ones((16,), jnp.float32)
        stage[pl.ds(i * 16, 16)] = jnp.zeros((16,), jnp.float32)
        return 0

    lax.fori_loop(0, BATCH // 16, _fill, 0, unroll=True)

    def _zero(i, _):
        stage[pl.ds(i * 16, 16)] = jnp.zeros((16,), jnp.float32)
        return 0

    lax.fori_loop(BATCH // 16, ROWS_PER_TILE // 16, _zero, 0)
    pltpu.sync_copy(stage, acc.at[pl.ds(sid * ROWS_PER_TILE, ROWS_PER_TILE)])
    pltpu.sync_copy(dst_hbm.at[pl.ds(tile * NCHUNK, NCHUNK)], didx)
    plsc.subcore_barrier()

    def _body(j, _):
        pltpu.sync_copy(ones_v, acc.at[didx.at[j]], add=True)
        return 0

    lax.fori_loop(0, NCHUNK, _body, 0)
    plsc.subcore_barrier()
    sl = pl.ds(sid * ROWS_PER_TILE, ROWS_PER_TILE)
    pltpu.sync_copy(acc.at[sl], out_hbm.at[cid, sl])


@functools.partial(
    pl.kernel,
    out_type=jax.ShapeDtypeStruct((NCORE, N_PAD, DH), jnp.float32),
    mesh=_mesh,
    scratch_types=[
        pltpu.VMEM_SHARED((N_PAD, DH), jnp.float32),
        pltpu.VMEM((NCHUNK, BATCH), jnp.int32),
        pltpu.VMEM((NCHUNK, BATCH), jnp.int32),
        pltpu.VMEM((BATCH, DH), jnp.float32),
        pltpu.VMEM((BATCH, DH), jnp.float32),
        pltpu.VMEM((ROWS_PER_TILE, DH), jnp.float32),
        pltpu.SemaphoreType.DMA,
        pltpu.SemaphoreType.DMA,
    ],
    compiler_params=_sc_params,
)
def _agg_kernel(hs_hbm, src_hbm, dst_hbm, out_hbm,
                acc, sidx, didx, rows0, rows1, stage, sem0, sem1):
    cid = lax.axis_index("c")
    sid = lax.axis_index("s")
    tile = cid * NSUB + sid

    def _zero(i, _):
        stage[i, :] = jnp.zeros((DH,), jnp.float32)
        return 0

    lax.fori_loop(0, ROWS_PER_TILE, _zero, 0, unroll=8)
    pltpu.sync_copy(stage, acc.at[pl.ds(sid * ROWS_PER_TILE, ROWS_PER_TILE)])
    pltpu.sync_copy(src_hbm.at[pl.ds(tile * NCHUNK, NCHUNK)], sidx)
    pltpu.sync_copy(dst_hbm.at[pl.ds(tile * NCHUNK, NCHUNK)], didx)
    plsc.subcore_barrier()

    pltpu.async_copy(hs_hbm.at[sidx.at[0]], rows0, sem0)
    pltpu.async_copy(hs_hbm.at[sidx.at[1]], rows1, sem1)

    def _body(k, _):
        j = 2 * k
        pltpu.make_async_copy(hs_hbm.at[sidx.at[j]], rows0, sem0).wait()
        pltpu.sync_copy(rows0, acc.at[didx.at[j]], add=True)
        pltpu.async_copy(hs_hbm.at[sidx.at[j + 2]], rows0, sem0)
        pltpu.make_async_copy(hs_hbm.at[sidx.at[j + 1]], rows1, sem1).wait()
        pltpu.sync_copy(rows1, acc.at[didx.at[j + 1]], add=True)
        pltpu.async_copy(hs_hbm.at[sidx.at[j + 3]], rows1, sem1)
        return 0

    lax.fori_loop(0, NCHUNK // 2 - 1, _body, 0)
    jT = NCHUNK - 2
    pltpu.make_async_copy(hs_hbm.at[sidx.at[jT]], rows0, sem0).wait()
    pltpu.sync_copy(rows0, acc.at[didx.at[jT]], add=True)
    pltpu.make_async_copy(hs_hbm.at[sidx.at[jT + 1]], rows1, sem1).wait()
    pltpu.sync_copy(rows1, acc.at[didx.at[jT + 1]], add=True)

    plsc.subcore_barrier()
    sl = pl.ds(sid * ROWS_PER_TILE, ROWS_PER_TILE)
    pltpu.sync_copy(acc.at[sl], out_hbm.at[cid, sl])


def _tc_a_body(x_ref, w1_ref, degp_ref, hs_ref):
    deg = degp_ref[:, 0:1] + degp_ref[:, 1:2] + 1.0
    dinv = lax.rsqrt(deg)
    h = jnp.dot(x_ref[...], w1_ref[...], preferred_element_type=jnp.float32)
    hs_ref[...] = h * dinv


_tc_a = pl.pallas_call(
    _tc_a_body,
    grid=(GRID,),
    in_specs=[
        pl.BlockSpec((NB, DF), lambda i: (i, 0)),
        pl.BlockSpec((DF, DH), lambda i: (0, 0)),
        pl.BlockSpec((NB, 2), lambda i: (i, 0)),
    ],
    out_specs=pl.BlockSpec((NB, DH), lambda i: (i, 0)),
    out_shape=jax.ShapeDtypeStruct((N, DH), jnp.float32),
)


def _tc_b_body(hs1_ref, agg_ref, degp_ref, w2_ref, b1_ref, hs2_ref):
    deg = degp_ref[:, 0:1] + degp_ref[:, 1:2] + 1.0
    dinv = lax.rsqrt(deg)
    agg = agg_ref[0] + agg_ref[1]
    out1 = (agg + hs1_ref[...]) * dinv + b1_ref[...]
    r = jnp.maximum(out1, 0.0)
    h2 = jnp.dot(r, w2_ref[...], preferred_element_type=jnp.float32)
    hs2_ref[...] = h2 * dinv


_tc_b = pl.pallas_call(
    _tc_b_body,
    grid=(GRID,),
    in_specs=[
        pl.BlockSpec((NB, DH), lambda i: (i, 0)),
        pl.BlockSpec((NCORE, NB, DH), lambda i: (0, i, 0)),
        pl.BlockSpec((NB, 2), lambda i: (i, 0)),
        pl.BlockSpec((DH, NCLS), lambda i: (0, 0)),
        pl.BlockSpec((1, DH), lambda i: (0, 0)),
    ],
    out_specs=pl.BlockSpec((NB, NCLS), lambda i: (i, 0)),
    out_shape=jax.ShapeDtypeStruct((N, NCLS), jnp.float32),
)


def _tc_c_body(hs2_ref, agg_ref, degp_ref, b2_ref, out_ref):
    deg = degp_ref[:, 0:1] + degp_ref[:, 1:2] + 1.0
    dinv = lax.rsqrt(deg)
    agg = agg_ref[0] + agg_ref[1]
    out_ref[...] = (agg + hs2_ref[...]) * dinv + b2_ref[...]


_tc_c = pl.pallas_call(
    _tc_c_body,
    grid=(GRID,),
    in_specs=[
        pl.BlockSpec((NB, NCLS), lambda i: (i, 0)),
        pl.BlockSpec((NCORE, NB, NCLS), lambda i: (0, i, 0)),
        pl.BlockSpec((NB, 2), lambda i: (i, 0)),
        pl.BlockSpec((1, NCLS), lambda i: (0, 0)),
    ],
    out_specs=pl.BlockSpec((NB, NCLS), lambda i: (i, 0)),
    out_shape=jax.ShapeDtypeStruct((N, NCLS), jnp.float32),
)


def kernel(x, edge_index, W1, b1, W2, b2):
    ei = edge_index.astype(jnp.int32)
    pad = E_PAD - E
    src_p = jnp.concatenate([ei[0], jnp.zeros((pad,), jnp.int32)])
    dst_p = jnp.concatenate([ei[1], jnp.full((pad,), N, jnp.int32)])
    src_p = src_p.reshape(NTILE * NCHUNK, BATCH)
    dst_p = dst_p.reshape(NTILE * NCHUNK, BATCH)

    degp = _deg_kernel(dst_p)
    degp_t = degp.T
    hs1 = _tc_a(x, W1, degp_t)
    agg1 = _agg_kernel(hs1, src_p, dst_p)
    hs2 = _tc_b(hs1, agg1, degp_t, W2, b1.reshape(1, DH))
    agg2 = _agg_kernel(hs2, src_p, dst_p)
    return _tc_c(hs2, agg2, degp_t, b2.reshape(1, NCLS))


# speedup vs baseline: 50.2963x; 1.2887x over previous
import functools

import jax
import jax.numpy as jnp
from jax import lax
from jax.experimental import pallas as pl
from jax.experimental.pallas import tpu as pltpu
from jax.experimental.pallas import tpu_sc as plsc

N = 10000
E = 320000
DF = 128
DH = 16
NCLS = 16

NCORE = 2
NSUB = 16
NTILE = NCORE * NSUB

N_PAD = 10240
ROWS_PER_TILE = N_PAD // NSUB
BATCH = 128
NCHUNKS = E // BATCH
TPC = 80
LAST_C = NCHUNKS - (NTILE - 1) * TPC

_mesh = plsc.VectorSubcoreMesh(core_axis_name="c", subcore_axis_name="s")
_sc_params = pltpu.CompilerParams(use_tc_tiling_on_sc=False)


@functools.partial(
    pl.kernel,
    out_type=jax.ShapeDtypeStruct((NCORE, N_PAD), jnp.float32),
    mesh=_mesh,
    scratch_types=[
        pltpu.VMEM_SHARED((N_PAD,), jnp.float32),
        pltpu.VMEM((TPC, BATCH), jnp.int32),
        pltpu.VMEM((BATCH,), jnp.float32),
        pltpu.VMEM((ROWS_PER_TILE,), jnp.float32),
        pltpu.SemaphoreType.DMA,
        pltpu.SemaphoreType.DMA,
    ],
    compiler_params=_sc_params,
)
def _deg_kernel(dst_hbm, out_hbm, acc, didx, ones_v, stage, s0, s1):
    cid = lax.axis_index("c")
    sid = lax.axis_index("s")
    tile = cid * NSUB + sid
    is_last = tile == NTILE - 1

    def _fill(i, _):
        ones_v[pl.ds(i * 16, 16)] = jnp.ones((16,), jnp.float32)
        return 0

    lax.fori_loop(0, BATCH // 16, _fill, 0, unroll=True)

    def _zero(i, _):
        stage[pl.ds(i * 16, 16)] = jnp.zeros((16,), jnp.float32)
        return 0

    lax.fori_loop(0, ROWS_PER_TILE // 16, _zero, 0)
    pltpu.sync_copy(stage, acc.at[pl.ds(sid * ROWS_PER_TILE, ROWS_PER_TILE)])

    @pl.when(jnp.logical_not(is_last))
    def _():
        pltpu.sync_copy(dst_hbm.at[pl.ds(tile * TPC, TPC)], didx)

    @pl.when(is_last)
    def _():
        pltpu.sync_copy(dst_hbm.at[pl.ds(tile * TPC, LAST_C)],
                        didx.at[pl.ds(0, LAST_C)])

    plsc.subcore_barrier()
    npair = jnp.where(is_last, LAST_C // 2, TPC // 2)

    def _body(k, _):
        j = 2 * k
        pltpu.async_copy(ones_v, acc.at[didx.at[j]], s0, add=True)
        pltpu.async_copy(ones_v, acc.at[didx.at[j + 1]], s1, add=True)
        pltpu.make_async_copy(ones_v, acc.at[didx.at[j]], s0).wait()
        pltpu.make_async_copy(ones_v, acc.at[didx.at[j + 1]], s1).wait()
        return 0

    lax.fori_loop(0, npair, _body, 0)
    plsc.subcore_barrier()
    sl = pl.ds(sid * ROWS_PER_TILE, ROWS_PER_TILE)
    pltpu.sync_copy(acc.at[sl], out_hbm.at[cid, sl])


@functools.partial(
    pl.kernel,
    out_type=jax.ShapeDtypeStruct((NCORE, N_PAD, DH), jnp.float32),
    mesh=_mesh,
    scratch_types=[
        pltpu.VMEM_SHARED((N_PAD, DH), jnp.float32),
        pltpu.VMEM((TPC, BATCH), jnp.int32),
        pltpu.VMEM((TPC, BATCH), jnp.int32),
        pltpu.VMEM((BATCH, DH), jnp.float32),
        pltpu.VMEM((BATCH, DH), jnp.float32),
        pltpu.VMEM((ROWS_PER_TILE, DH), jnp.float32),
        pltpu.SemaphoreType.DMA,
        pltpu.SemaphoreType.DMA,
        pltpu.SemaphoreType.DMA,
        pltpu.SemaphoreType.DMA,
    ],
    compiler_params=_sc_params,
)
def _agg_kernel(hs_hbm, src_hbm, dst_hbm, out_hbm,
                acc, sidx, didx, rows0, rows1, stage, g0, g1, s0, s1):
    cid = lax.axis_index("c")
    sid = lax.axis_index("s")
    tile = cid * NSUB + sid
    is_last = tile == NTILE - 1

    def _zero(i, _):
        stage[i, :] = jnp.zeros((DH,), jnp.float32)
        return 0

    lax.fori_loop(0, ROWS_PER_TILE, _zero, 0, unroll=8)
    pltpu.sync_copy(stage, acc.at[pl.ds(sid * ROWS_PER_TILE, ROWS_PER_TILE)])

    @pl.when(jnp.logical_not(is_last))
    def _():
        pltpu.sync_copy(src_hbm.at[pl.ds(tile * TPC, TPC)], sidx)
        pltpu.sync_copy(dst_hbm.at[pl.ds(tile * TPC, TPC)], didx)

    @pl.when(is_last)
    def _():
        pltpu.sync_copy(src_hbm.at[pl.ds(tile * TPC, LAST_C)],
                        sidx.at[pl.ds(0, LAST_C)])
        pltpu.sync_copy(dst_hbm.at[pl.ds(tile * TPC, LAST_C)],
                        didx.at[pl.ds(0, LAST_C)])

    plsc.subcore_barrier()
    nch = jnp.where(is_last, LAST_C, TPC)

    pltpu.async_copy(hs_hbm.at[sidx.at[0]], rows0, g0)
    pltpu.async_copy(hs_hbm.at[sidx.at[1]], rows1, g1)

    def _body(k, _):
        j = 2 * k
        pltpu.make_async_copy(hs_hbm.at[sidx.at[j]], rows0, g0).wait()
        pltpu.async_copy(rows0, acc.at[didx.at[j]], s0, add=True)
        pltpu.make_async_copy(hs_hbm.at[sidx.at[j + 1]], rows1, g1).wait()
        pltpu.async_copy(rows1, acc.at[didx.at[j + 1]], s1, add=True)
        pltpu.make_async_copy(rows0, acc.at[didx.at[j]], s0).wait()
        pltpu.async_copy(hs_hbm.at[sidx.at[j + 2]], rows0, g0)
        pltpu.make_async_copy(rows1, acc.at[didx.at[j + 1]], s1).wait()
        pltpu.async_copy(hs_hbm.at[sidx.at[j + 3]], rows1, g1)
        return 0

    lax.fori_loop(0, nch // 2 - 1, _body, 0)
    jT = nch - 2
    pltpu.make_async_copy(hs_hbm.at[sidx.at[jT]], rows0, g0).wait()
    pltpu.sync_copy(rows0, acc.at[didx.at[jT]], add=True)
    pltpu.make_async_copy(hs_hbm.at[sidx.at[jT + 1]], rows1, g1).wait()
    pltpu.sync_copy(rows1, acc.at[didx.at[jT + 1]], add=True)

    plsc.subcore_barrier()
    sl = pl.ds(sid * ROWS_PER_TILE, ROWS_PER_TILE)
    pltpu.sync_copy(acc.at[sl], out_hbm.at[cid, sl])


def _tc_mm_body(x_ref, w1_ref, h_ref):
    h_ref[...] = jnp.dot(x_ref[...], w1_ref[...],
                         preferred_element_type=jnp.float32)


_tc_mm = pl.pallas_call(
    _tc_mm_body,
    out_shape=jax.ShapeDtypeStruct((N, DH), jnp.float32),
)


def _dinv_of(degp_ref):
    deg = degp_ref[:N, 0:1] + degp_ref[:N, 1:2] + 1.0
    return lax.rsqrt(deg)


def _tc_scale_body(h_ref, degp_ref, hs_ref):
    hs_ref[...] = h_ref[...] * _dinv_of(degp_ref)


_tc_scale = pl.pallas_call(
    _tc_scale_body,
    out_shape=jax.ShapeDtypeStruct((N, DH), jnp.float32),
)


def _tc_b_body(hs1_ref, agg_ref, degp_ref, w2_ref, b1_ref, hs2_ref):
    dinv = _dinv_of(degp_ref)
    agg = agg_ref[0, :N] + agg_ref[1, :N]
    out1 = (agg + hs1_ref[...]) * dinv + b1_ref[...]
    r = jnp.maximum(out1, 0.0)
    h2 = jnp.dot(r, w2_ref[...], preferred_element_type=jnp.float32)
    hs2_ref[...] = h2 * dinv


_tc_b = pl.pallas_call(
    _tc_b_body,
    out_shape=jax.ShapeDtypeStruct((N, NCLS), jnp.float32),
)


def _tc_c_body(hs2_ref, agg_ref, degp_ref, b2_ref, out_ref):
    dinv = _dinv_of(degp_ref)
    agg = agg_ref[0, :N] + agg_ref[1, :N]
    out_ref[...] = (agg + hs2_ref[...]) * dinv + b2_ref[...]


_tc_c = pl.pallas_call(
    _tc_c_body,
    out_shape=jax.ShapeDtypeStruct((N, NCLS), jnp.float32),
)


def kernel(x, edge_index, W1, b1, W2, b2):
    ei = edge_index.astype(jnp.int32).reshape(2, NCHUNKS, BATCH)
    src2d = ei[0]
    dst2d = ei[1]

    h1 = _tc_mm(x, W1)
    degp = _deg_kernel(dst2d)
    degp_t = degp.T
    hs1 = _tc_scale(h1, degp_t)
    agg1 = _agg_kernel(hs1, src2d, dst2d)
    hs2 = _tc_b(hs1, agg1, degp_t, W2, b1.reshape(1, DH))
    agg2 = _agg_kernel(hs2, src2d, dst2d)
    return _tc_c(hs2, agg2, degp_t, b2.reshape(1, NCLS))


# speedup vs baseline: 59.2738x; 1.1785x over previous
import functools

import jax
import jax.numpy as jnp
from jax import lax
from jax.experimental import pallas as pl
from jax.experimental.pallas import tpu as pltpu
from jax.experimental.pallas import tpu_sc as plsc

N = 10000
E = 320000
DF = 128
DH = 16
NCLS = 16

NCORE = 2
NSUB = 16
NTILE = NCORE * NSUB

N_PAD = 10240
ROWS_PER_TILE = N_PAD // NSUB
BATCH = 128
NCHUNKS = E // BATCH
TPC = 80
LAST_C = NCHUNKS - (NTILE - 1) * TPC

NPK = N // 8
NPK_PAD = N_PAD // 8

_mesh = plsc.VectorSubcoreMesh(core_axis_name="c", subcore_axis_name="s")
_sc_params = pltpu.CompilerParams(use_tc_tiling_on_sc=False)


@functools.partial(
    pl.kernel,
    out_type=jax.ShapeDtypeStruct((NCORE, N_PAD), jnp.float32),
    mesh=_mesh,
    scratch_types=[
        pltpu.VMEM_SHARED((N_PAD,), jnp.float32),
        pltpu.VMEM((TPC, BATCH), jnp.int32),
        pltpu.VMEM((BATCH,), jnp.float32),
        pltpu.VMEM((ROWS_PER_TILE,), jnp.float32),
        pltpu.SemaphoreType.DMA,
        pltpu.SemaphoreType.DMA,
    ],
    compiler_params=_sc_params,
)
def _deg_kernel(src_hbm, dst_hbm, out_hbm, acc, didx, ones_v, stage, s0, s1):
    del src_hbm
    cid = lax.axis_index("c")
    sid = lax.axis_index("s")
    tile = cid * NSUB + sid
    is_last = tile == NTILE - 1

    def _fill(i, _):
        ones_v[pl.ds(i * 16, 16)] = jnp.ones((16,), jnp.float32)
        return 0

    lax.fori_loop(0, BATCH // 16, _fill, 0, unroll=True)

    def _zero(i, _):
        stage[pl.ds(i * 16, 16)] = jnp.zeros((16,), jnp.float32)
        return 0

    lax.fori_loop(0, ROWS_PER_TILE // 16, _zero, 0)
    pltpu.sync_copy(stage, acc.at[pl.ds(sid * ROWS_PER_TILE, ROWS_PER_TILE)])

    @pl.when(jnp.logical_not(is_last))
    def _():
        pltpu.sync_copy(dst_hbm.at[pl.ds(tile * TPC, TPC)], didx)

    @pl.when(is_last)
    def _():
        pltpu.sync_copy(dst_hbm.at[pl.ds(tile * TPC, LAST_C)],
                        didx.at[pl.ds(0, LAST_C)])

    plsc.subcore_barrier()
    npair = jnp.where(is_last, LAST_C // 2, TPC // 2)

    def _body(k, _):
        j = 2 * k
        pltpu.async_copy(ones_v, acc.at[didx.at[j]], s0, add=True)
        pltpu.async_copy(ones_v, acc.at[didx.at[j + 1]], s1, add=True)
        pltpu.make_async_copy(ones_v, acc.at[didx.at[j]], s0).wait()
        pltpu.make_async_copy(ones_v, acc.at[didx.at[j + 1]], s1).wait()
        return 0

    lax.fori_loop(0, npair, _body, 0)
    plsc.subcore_barrier()
    sl = pl.ds(sid * ROWS_PER_TILE, ROWS_PER_TILE)
    pltpu.sync_copy(acc.at[sl], out_hbm.at[cid, sl])


@functools.partial(
    pl.kernel,
    out_type=jax.ShapeDtypeStruct((NCORE, N_PAD, DH), jnp.float32),
    mesh=_mesh,
    scratch_types=[
        pltpu.VMEM_SHARED((N_PAD, DH), jnp.float32),
        pltpu.VMEM((TPC, BATCH), jnp.int32),
        pltpu.VMEM((TPC, BATCH), jnp.int32),
        pltpu.VMEM((BATCH, DH), jnp.float32),
        pltpu.VMEM((BATCH, DH), jnp.float32),
        pltpu.VMEM((ROWS_PER_TILE, DH), jnp.float32),
        pltpu.SemaphoreType.DMA,
        pltpu.SemaphoreType.DMA,
        pltpu.SemaphoreType.DMA,
        pltpu.SemaphoreType.DMA,
    ],
    compiler_params=_sc_params,
)
def _agg_kernel(hs_hbm, src_hbm, dst_hbm, out_hbm,
                acc, sidx, didx, rows0, rows1, stage, g0, g1, s0, s1):
    cid = lax.axis_index("c")
    sid = lax.axis_index("s")
    tile = cid * NSUB + sid
    is_last = tile == NTILE - 1

    def _zero(i, _):
        stage[i, :] = jnp.zeros((DH,), jnp.float32)
        return 0

    lax.fori_loop(0, ROWS_PER_TILE, _zero, 0, unroll=8)
    pltpu.sync_copy(stage, acc.at[pl.ds(sid * ROWS_PER_TILE, ROWS_PER_TILE)])

    @pl.when(jnp.logical_not(is_last))
    def _():
        pltpu.sync_copy(src_hbm.at[pl.ds(tile * TPC, TPC)], sidx)
        pltpu.sync_copy(dst_hbm.at[pl.ds(tile * TPC, TPC)], didx)

    @pl.when(is_last)
    def _():
        pltpu.sync_copy(src_hbm.at[pl.ds(tile * TPC, LAST_C)],
                        sidx.at[pl.ds(0, LAST_C)])
        pltpu.sync_copy(dst_hbm.at[pl.ds(tile * TPC, LAST_C)],
                        didx.at[pl.ds(0, LAST_C)])

    plsc.subcore_barrier()
    nch = jnp.where(is_last, LAST_C, TPC)

    pltpu.async_copy(hs_hbm.at[sidx.at[0]], rows0, g0)
    pltpu.async_copy(hs_hbm.at[sidx.at[1]], rows1, g1)

    def _body(k, _):
        j = 2 * k
        pltpu.make_async_copy(hs_hbm.at[sidx.at[j]], rows0, g0).wait()
        pltpu.async_copy(rows0, acc.at[didx.at[j]], s0, add=True)
        pltpu.make_async_copy(hs_hbm.at[sidx.at[j + 1]], rows1, g1).wait()
        pltpu.async_copy(rows1, acc.at[didx.at[j + 1]], s1, add=True)
        pltpu.make_async_copy(rows0, acc.at[didx.at[j]], s0).wait()
        pltpu.async_copy(hs_hbm.at[sidx.at[j + 2]], rows0, g0)
        pltpu.make_async_copy(rows1, acc.at[didx.at[j + 1]], s1).wait()
        pltpu.async_copy(hs_hbm.at[sidx.at[j + 3]], rows1, g1)
        return 0

    lax.fori_loop(0, nch // 2 - 1, _body, 0)
    jT = nch - 2
    pltpu.make_async_copy(hs_hbm.at[sidx.at[jT]], rows0, g0).wait()
    pltpu.sync_copy(rows0, acc.at[didx.at[jT]], add=True)
    pltpu.make_async_copy(hs_hbm.at[sidx.at[jT + 1]], rows1, g1).wait()
    pltpu.sync_copy(rows1, acc.at[didx.at[jT + 1]], add=True)

    plsc.subcore_barrier()
    sl = pl.ds(sid * ROWS_PER_TILE, ROWS_PER_TILE)
    pltpu.sync_copy(acc.at[sl], out_hbm.at[cid, sl])


def _tc_mm_body(xr_ref, w1bd_ref, h_ref):
    h_ref[...] = jnp.dot(xr_ref[...], w1bd_ref[...],
                         preferred_element_type=jnp.float32)


_tc_mm = pl.pallas_call(
    _tc_mm_body,
    out_shape=jax.ShapeDtypeStruct((NPK, 8 * DH), jnp.float32),
)


def _tc_scale_body(h_ref, deg16_ref, dinv16_ref, hs_ref):
    deg = deg16_ref[0] + deg16_ref[1] + 1.0
    dinv16 = lax.rsqrt(deg)
    dinv16_ref[...] = dinv16
    hs_ref[...] = h_ref[...] * dinv16[:NPK]


_tc_scale = pl.pallas_call(
    _tc_scale_body,
    out_shape=(
        jax.ShapeDtypeStruct((NPK_PAD, 8 * DH), jnp.float32),
        jax.ShapeDtypeStruct((NPK, 8 * DH), jnp.float32),
    ),
)


def _tc_b_body(hs1_ref, agg_ref, dinv16_ref, w2bd_ref, b1_ref, hs2_ref):
    dinv16 = dinv16_ref[:NPK]
    agg = agg_ref[0, :NPK] + agg_ref[1, :NPK]
    out1 = (agg + hs1_ref[...]) * dinv16 + b1_ref[...]
    r = jnp.maximum(out1, 0.0)
    h2 = jnp.dot(r, w2bd_ref[...], preferred_element_type=jnp.float32)
    hs2_ref[...] = h2 * dinv16


_tc_b = pl.pallas_call(
    _tc_b_body,
    out_shape=jax.ShapeDtypeStruct((NPK, 8 * NCLS), jnp.float32),
)


def _tc_c_body(hs2_ref, agg_ref, dinv16_ref, b2_ref, out_ref):
    agg = agg_ref[0, :NPK] + agg_ref[1, :NPK]
    out_ref[...] = (agg + hs2_ref[...]) * dinv16_ref[:NPK] + b2_ref[...]


_tc_c = pl.pallas_call(
    _tc_c_body,
    out_shape=jax.ShapeDtypeStruct((NPK, 8 * NCLS), jnp.float32),
)


def _block_diag8(w):
    a, b = w.shape
    z = jnp.zeros((a, b), w.dtype)
    rows = [jnp.concatenate([w if i == j else z for j in range(8)], axis=1)
            for i in range(8)]
    return jnp.concatenate(rows, axis=0)


def kernel(x, edge_index, W1, b1, W2, b2):
    ei = edge_index.astype(jnp.int32).reshape(2, NCHUNKS, BATCH)
    src2d = ei[0]
    dst2d = ei[1]

    w1bd = _block_diag8(W1)
    w2bd = _block_diag8(W2)
    b1t = jnp.tile(b1, 8).reshape(1, 8 * DH)
    b2t = jnp.tile(b2, 8).reshape(1, 8 * NCLS)
    xr = x.reshape(NPK, 8 * DF)

    h1p = _tc_mm(xr, w1bd)
    degp = _deg_kernel(src2d, dst2d)
    deg16 = jnp.broadcast_to(degp[:, :, None], (NCORE, N_PAD, DH))
    deg16p = deg16.reshape(NCORE, NPK_PAD, 8 * DH)
    dinv16, hs1p = _tc_scale(h1p, deg16p)
    hs1 = hs1p.reshape(N, DH)
    agg1 = _agg_kernel(hs1, src2d, dst2d)
    agg1p = agg1.reshape(NCORE, NPK_PAD, 8 * DH)
    hs2p = _tc_b(hs1p, agg1p, dinv16, w2bd, b1t)
    hs2 = hs2p.reshape(N, DH)
    agg2 = _agg_kernel(hs2, src2d, dst2d)
    agg2p = agg2.reshape(NCORE, NPK_PAD, 8 * NCLS)
    outp = _tc_c(hs2p, agg2p, dinv16, b2t)
    return outp.reshape(N, NCLS)


# speedup vs baseline: 64.8240x; 1.0936x over previous
import functools

import jax
import jax.numpy as jnp
from jax import lax
from jax.experimental import pallas as pl
from jax.experimental.pallas import tpu as pltpu
from jax.experimental.pallas import tpu_sc as plsc

N = 10000
E = 320000
DF = 128
DH = 16
NCLS = 16

NCORE = 2
NSUB = 16
NTILE = NCORE * NSUB

N_PAD = 10240
ROWS_PER_TILE = N_PAD // NSUB
BATCH = 128
NCHUNKS = E // BATCH
TPC = 80
LAST_C = NCHUNKS - (NTILE - 1) * TPC

NPK = N // 8
NPK_PAD = N_PAD // 8

_mesh = plsc.VectorSubcoreMesh(core_axis_name="c", subcore_axis_name="s")
_sc_params = pltpu.CompilerParams(use_tc_tiling_on_sc=False)


@functools.partial(
    pl.kernel,
    out_type=jax.ShapeDtypeStruct((NCORE, N_PAD), jnp.float32),
    mesh=_mesh,
    scratch_types=[
        pltpu.VMEM_SHARED((N_PAD,), jnp.float32),
        pltpu.VMEM((TPC, BATCH), jnp.int32),
        pltpu.VMEM((BATCH,), jnp.float32),
        pltpu.VMEM((ROWS_PER_TILE,), jnp.float32),
        pltpu.SemaphoreType.DMA,
        pltpu.SemaphoreType.DMA,
    ],
    compiler_params=_sc_params,
)
def _deg_kernel(src_hbm, dst_hbm, out_hbm, acc, didx, ones_v, stage, s0, s1):
    del src_hbm
    cid = lax.axis_index("c")
    sid = lax.axis_index("s")
    tile = cid * NSUB + sid
    is_last = tile == NTILE - 1

    def _fill(i, _):
        ones_v[pl.ds(i * 16, 16)] = jnp.ones((16,), jnp.float32)
        return 0

    lax.fori_loop(0, BATCH // 16, _fill, 0, unroll=True)

    def _zero(i, _):
        stage[pl.ds(i * 16, 16)] = jnp.zeros((16,), jnp.float32)
        return 0

    lax.fori_loop(0, ROWS_PER_TILE // 16, _zero, 0)
    pltpu.sync_copy(stage, acc.at[pl.ds(sid * ROWS_PER_TILE, ROWS_PER_TILE)])

    @pl.when(jnp.logical_not(is_last))
    def _():
        pltpu.sync_copy(dst_hbm.at[pl.ds(tile * TPC, TPC)], didx)

    @pl.when(is_last)
    def _():
        pltpu.sync_copy(dst_hbm.at[pl.ds(tile * TPC, LAST_C)],
                        didx.at[pl.ds(0, LAST_C)])

    plsc.subcore_barrier()
    npair = jnp.where(is_last, LAST_C // 2, TPC // 2)

    def _body(k, _):
        j = 2 * k
        pltpu.async_copy(ones_v, acc.at[didx.at[j]], s0, add=True)
        pltpu.async_copy(ones_v, acc.at[didx.at[j + 1]], s1, add=True)
        pltpu.make_async_copy(ones_v, acc.at[didx.at[j]], s0).wait()
        pltpu.make_async_copy(ones_v, acc.at[didx.at[j + 1]], s1).wait()
        return 0

    lax.fori_loop(0, npair, _body, 0)
    plsc.subcore_barrier()
    sl = pl.ds(sid * ROWS_PER_TILE, ROWS_PER_TILE)
    pltpu.sync_copy(acc.at[sl], out_hbm.at[cid, sl])


@functools.partial(
    pl.kernel,
    out_type=jax.ShapeDtypeStruct((NCORE, N_PAD, DH), jnp.float32),
    mesh=_mesh,
    scratch_types=[
        pltpu.VMEM_SHARED((N_PAD, DH), jnp.float32),
        pltpu.VMEM((TPC, BATCH), jnp.int32),
        pltpu.VMEM((TPC, BATCH), jnp.int32),
        pltpu.VMEM((BATCH, DH), jnp.float32),
        pltpu.VMEM((BATCH, DH), jnp.float32),
        pltpu.VMEM((ROWS_PER_TILE, DH), jnp.float32),
        pltpu.SemaphoreType.DMA,
        pltpu.SemaphoreType.DMA,
        pltpu.SemaphoreType.DMA,
        pltpu.SemaphoreType.DMA,
    ],
    compiler_params=_sc_params,
)
def _agg_kernel(hs_hbm, src_hbm, dst_hbm, out_hbm,
                acc, sidx, didx, rows0, rows1, stage, g0, g1, s0, s1):
    cid = lax.axis_index("c")
    sid = lax.axis_index("s")
    tile = cid * NSUB + sid
    is_last = tile == NTILE - 1

    def _zero(i, _):
        stage[i, :] = jnp.zeros((DH,), jnp.float32)
        return 0

    lax.fori_loop(0, ROWS_PER_TILE, _zero, 0, unroll=8)
    pltpu.sync_copy(stage, acc.at[pl.ds(sid * ROWS_PER_TILE, ROWS_PER_TILE)])

    @pl.when(jnp.logical_not(is_last))
    def _():
        pltpu.sync_copy(src_hbm.at[pl.ds(tile * TPC, TPC)], sidx)
        pltpu.sync_copy(dst_hbm.at[pl.ds(tile * TPC, TPC)], didx)

    @pl.when(is_last)
    def _():
        pltpu.sync_copy(src_hbm.at[pl.ds(tile * TPC, LAST_C)],
                        sidx.at[pl.ds(0, LAST_C)])
        pltpu.sync_copy(dst_hbm.at[pl.ds(tile * TPC, LAST_C)],
                        didx.at[pl.ds(0, LAST_C)])

    plsc.subcore_barrier()
    nch = jnp.where(is_last, LAST_C, TPC)

    pltpu.async_copy(hs_hbm.at[sidx.at[0]], rows0, g0)
    pltpu.async_copy(hs_hbm.at[sidx.at[1]], rows1, g1)

    def _body(k, _):
        j = 2 * k
        pltpu.make_async_copy(hs_hbm.at[sidx.at[j]], rows0, g0).wait()
        pltpu.async_copy(rows0, acc.at[didx.at[j]], s0, add=True)
        pltpu.make_async_copy(hs_hbm.at[sidx.at[j + 1]], rows1, g1).wait()
        pltpu.async_copy(rows1, acc.at[didx.at[j + 1]], s1, add=True)
        pltpu.make_async_copy(rows0, acc.at[didx.at[j]], s0).wait()
        pltpu.async_copy(hs_hbm.at[sidx.at[j + 2]], rows0, g0)
        pltpu.make_async_copy(rows1, acc.at[didx.at[j + 1]], s1).wait()
        pltpu.async_copy(hs_hbm.at[sidx.at[j + 3]], rows1, g1)
        return 0

    lax.fori_loop(0, nch // 2 - 1, _body, 0)
    jT = nch - 2
    pltpu.make_async_copy(hs_hbm.at[sidx.at[jT]], rows0, g0).wait()
    pltpu.sync_copy(rows0, acc.at[didx.at[jT]], add=True)
    pltpu.make_async_copy(hs_hbm.at[sidx.at[jT + 1]], rows1, g1).wait()
    pltpu.sync_copy(rows1, acc.at[didx.at[jT + 1]], add=True)

    plsc.subcore_barrier()
    sl = pl.ds(sid * ROWS_PER_TILE, ROWS_PER_TILE)
    pltpu.sync_copy(acc.at[sl], out_hbm.at[cid, sl])


def _tc_mm_body(xr_ref, w1bd_ref, h_ref):
    h_ref[...] = jnp.dot(xr_ref[...], w1bd_ref[...],
                         preferred_element_type=jnp.float32)


_tc_mm = pl.pallas_call(
    _tc_mm_body,
    out_shape=jax.ShapeDtypeStruct((NPK, 8 * DH), jnp.float32),
)


def _tc_scale_body(h_ref, deg8_ref, dinv16_ref, hs_ref):
    sel = jnp.repeat(jnp.eye(8, dtype=jnp.float32), 16, axis=1)
    deg = deg8_ref[0] + deg8_ref[1]
    deg16 = jnp.dot(deg, sel, preferred_element_type=jnp.float32) + 1.0
    dinv16 = lax.rsqrt(deg16)
    dinv16_ref[...] = dinv16
    hs_ref[...] = h_ref[...] * dinv16[:NPK]


_tc_scale = pl.pallas_call(
    _tc_scale_body,
    out_shape=(
        jax.ShapeDtypeStruct((NPK_PAD, 8 * DH), jnp.float32),
        jax.ShapeDtypeStruct((NPK, 8 * DH), jnp.float32),
    ),
)


def _tc_b_body(hs1_ref, agg_ref, dinv16_ref, w2bd_ref, b1_ref, hs2_ref):
    dinv16 = dinv16_ref[:NPK]
    agg = agg_ref[0, :NPK] + agg_ref[1, :NPK]
    out1 = (agg + hs1_ref[...]) * dinv16 + b1_ref[...]
    r = jnp.maximum(out1, 0.0)
    h2 = jnp.dot(r, w2bd_ref[...], preferred_element_type=jnp.float32)
    hs2_ref[...] = h2 * dinv16


_tc_b = pl.pallas_call(
    _tc_b_body,
    out_shape=jax.ShapeDtypeStruct((NPK, 8 * NCLS), jnp.float32),
)


def _tc_c_body(hs2_ref, agg_ref, dinv16_ref, b2_ref, out_ref):
    agg = agg_ref[0, :NPK] + agg_ref[1, :NPK]
    out_ref[...] = (agg + hs2_ref[...]) * dinv16_ref[:NPK] + b2_ref[...]


_tc_c = pl.pallas_call(
    _tc_c_body,
    out_shape=jax.ShapeDtypeStruct((NPK, 8 * NCLS), jnp.float32),
)


def _block_diag8(w):
    a, b = w.shape
    z = jnp.zeros((a, b), w.dtype)
    rows = [jnp.concatenate([w if i == j else z for j in range(8)], axis=1)
            for i in range(8)]
    return jnp.concatenate(rows, axis=0)


def kernel(x, edge_index, W1, b1, W2, b2):
    ei = edge_index.astype(jnp.int32).reshape(2, NCHUNKS, BATCH)
    src2d = ei[0]
    dst2d = ei[1]

    w1bd = _block_diag8(W1)
    w2bd = _block_diag8(W2)
    b1t = jnp.tile(b1, 8).reshape(1, 8 * DH)
    b2t = jnp.tile(b2, 8).reshape(1, 8 * NCLS)
    xr = x.reshape(NPK, 8 * DF)

    h1p = _tc_mm(xr, w1bd)
    degp = _deg_kernel(src2d, dst2d)
    deg8 = degp.reshape(NCORE, NPK_PAD, 8)
    dinv16, hs1p = _tc_scale(h1p, deg8)
    hs1 = hs1p.reshape(N, DH)
    agg1 = _agg_kernel(hs1, src2d, dst2d)
    agg1p = agg1.reshape(NCORE, NPK_PAD, 8 * DH)
    hs2p = _tc_b(hs1p, agg1p, dinv16, w2bd, b1t)
    hs2 = hs2p.reshape(N, DH)
    agg2 = _agg_kernel(hs2, src2d, dst2d)
    agg2p = agg2.reshape(NCORE, NPK_PAD, 8 * NCLS)
    outp = _tc_c(hs2p, agg2p, dinv16, b2t)
    return outp.reshape(N, NCLS)


# speedup vs baseline: 81.7984x; 1.2619x over previous
import functools

import jax
import jax.numpy as jnp
from jax import lax
from jax.experimental import pallas as pl
from jax.experimental.pallas import tpu as pltpu
from jax.experimental.pallas import tpu_sc as plsc

N = 10000
E = 320000
DF = 128
DH = 16
NCLS = 16

NCORE = 2
NSUB = 16
NTILE = NCORE * NSUB

N_PAD = 10240
ROWS_PER_TILE = N_PAD // NSUB
BATCH = 128
NCHUNKS = E // BATCH
TPC = 80
LAST_C = NCHUNKS - (NTILE - 1) * TPC

NPK = N // 8
NPK_PAD = N_PAD // 8

_mesh = plsc.VectorSubcoreMesh(core_axis_name="c", subcore_axis_name="s")
_sc_params = pltpu.CompilerParams(use_tc_tiling_on_sc=False)


@functools.partial(
    pl.kernel,
    out_type=jax.ShapeDtypeStruct((NCORE, N_PAD), jnp.float32),
    mesh=_mesh,
    scratch_types=[
        pltpu.VMEM_SHARED((N_PAD,), jnp.float32),
        pltpu.VMEM((TPC, BATCH), jnp.int32),
        pltpu.VMEM((BATCH,), jnp.float32),
        pltpu.VMEM((ROWS_PER_TILE,), jnp.float32),
        pltpu.SemaphoreType.DMA,
        pltpu.SemaphoreType.DMA,
    ],
    compiler_params=_sc_params,
)
def _deg_kernel(src_hbm, dst_hbm, out_hbm, acc, didx, ones_v, stage, s0, s1):
    del src_hbm
    cid = lax.axis_index("c")
    sid = lax.axis_index("s")
    tile = cid * NSUB + sid
    is_last = tile == NTILE - 1

    def _fill(i, _):
        ones_v[pl.ds(i * 16, 16)] = jnp.ones((16,), jnp.float32)
        return 0

    lax.fori_loop(0, BATCH // 16, _fill, 0, unroll=True)

    def _zero(i, _):
        stage[pl.ds(i * 16, 16)] = jnp.zeros((16,), jnp.float32)
        return 0

    lax.fori_loop(0, ROWS_PER_TILE // 16, _zero, 0)
    pltpu.sync_copy(stage, acc.at[pl.ds(sid * ROWS_PER_TILE, ROWS_PER_TILE)])

    @pl.when(jnp.logical_not(is_last))
    def _():
        pltpu.sync_copy(dst_hbm.at[pl.ds(tile * TPC, TPC)], didx)

    @pl.when(is_last)
    def _():
        pltpu.sync_copy(dst_hbm.at[pl.ds(tile * TPC, LAST_C)],
                        didx.at[pl.ds(0, LAST_C)])

    plsc.subcore_barrier()
    npair = jnp.where(is_last, LAST_C // 2, TPC // 2)

    def _body(k, _):
        j = 2 * k
        pltpu.async_copy(ones_v, acc.at[didx.at[j]], s0, add=True)
        pltpu.async_copy(ones_v, acc.at[didx.at[j + 1]], s1, add=True)
        pltpu.make_async_copy(ones_v, acc.at[didx.at[j]], s0).wait()
        pltpu.make_async_copy(ones_v, acc.at[didx.at[j + 1]], s1).wait()
        return 0

    lax.fori_loop(0, npair, _body, 0)
    plsc.subcore_barrier()
    sl = pl.ds(sid * ROWS_PER_TILE, ROWS_PER_TILE)
    pltpu.sync_copy(acc.at[sl], out_hbm.at[cid, sl])


@functools.partial(
    pl.kernel,
    out_type=jax.ShapeDtypeStruct((NCORE, N_PAD, DH), jnp.float32),
    mesh=_mesh,
    scratch_types=[
        pltpu.VMEM_SHARED((N_PAD, DH), jnp.float32),
        pltpu.VMEM((TPC, BATCH), jnp.int32),
        pltpu.VMEM((TPC, BATCH), jnp.int32),
        pltpu.VMEM((BATCH, DH), jnp.float32),
        pltpu.VMEM((BATCH, DH), jnp.float32),
        pltpu.VMEM((BATCH, DH), jnp.float32),
        pltpu.VMEM((BATCH, DH), jnp.float32),
        pltpu.VMEM((ROWS_PER_TILE, DH), jnp.float32),
        pltpu.SemaphoreType.DMA,
        pltpu.SemaphoreType.DMA,
        pltpu.SemaphoreType.DMA,
        pltpu.SemaphoreType.DMA,
        pltpu.SemaphoreType.DMA,
        pltpu.SemaphoreType.DMA,
        pltpu.SemaphoreType.DMA,
        pltpu.SemaphoreType.DMA,
    ],
    compiler_params=_sc_params,
)
def _agg_kernel(hs_hbm, src_hbm, dst_hbm, out_hbm,
                acc, sidx, didx, rows0, rows1, rows2, rows3, stage,
                g0, g1, g2, g3, s0, s1, s2, s3):
    cid = lax.axis_index("c")
    sid = lax.axis_index("s")
    tile = cid * NSUB + sid
    is_last = tile == NTILE - 1

    def _zero(i, _):
        stage[i, :] = jnp.zeros((DH,), jnp.float32)
        return 0

    lax.fori_loop(0, ROWS_PER_TILE, _zero, 0, unroll=8)
    pltpu.sync_copy(stage, acc.at[pl.ds(sid * ROWS_PER_TILE, ROWS_PER_TILE)])

    @pl.when(jnp.logical_not(is_last))
    def _():
        pltpu.sync_copy(src_hbm.at[pl.ds(tile * TPC, TPC)], sidx)
        pltpu.sync_copy(dst_hbm.at[pl.ds(tile * TPC, TPC)], didx)

    @pl.when(is_last)
    def _():
        pltpu.sync_copy(src_hbm.at[pl.ds(tile * TPC, LAST_C)],
                        sidx.at[pl.ds(0, LAST_C)])
        pltpu.sync_copy(dst_hbm.at[pl.ds(tile * TPC, LAST_C)],
                        didx.at[pl.ds(0, LAST_C)])

    plsc.subcore_barrier()
    nch = jnp.where(is_last, LAST_C, TPC)
    rows = (rows0, rows1, rows2, rows3)
    gsem = (g0, g1, g2, g3)
    ssem = (s0, s1, s2, s3)

    for b in range(4):
        pltpu.async_copy(hs_hbm.at[sidx.at[b]], rows[b], gsem[b])

    def _body(k, _):
        j = 4 * k
        for b in range(4):
            pltpu.make_async_copy(
                hs_hbm.at[sidx.at[j + b]], rows[b], gsem[b]).wait()
            pltpu.async_copy(rows[b], acc.at[didx.at[j + b]], ssem[b],
                             add=True)
        for b in range(4):
            pltpu.make_async_copy(rows[b], acc.at[didx.at[j + b]],
                                  ssem[b]).wait()
            pltpu.async_copy(hs_hbm.at[sidx.at[j + 4 + b]], rows[b], gsem[b])
        return 0

    lax.fori_loop(0, nch // 4 - 1, _body, 0)
    jT = nch - 4
    for b in range(4):
        pltpu.make_async_copy(hs_hbm.at[sidx.at[jT + b]], rows[b],
                              gsem[b]).wait()
        pltpu.sync_copy(rows[b], acc.at[didx.at[jT + b]], add=True)

    plsc.subcore_barrier()
    sl = pl.ds(sid * ROWS_PER_TILE, ROWS_PER_TILE)
    pltpu.sync_copy(acc.at[sl], out_hbm.at[cid, sl])


def _tc_mm_body(xr_ref, w1bd_ref, h_ref):
    h_ref[...] = jnp.dot(xr_ref[...], w1bd_ref[...],
                         preferred_element_type=jnp.float32)


_tc_mm = pl.pallas_call(
    _tc_mm_body,
    out_shape=jax.ShapeDtypeStruct((NPK, 8 * DH), jnp.float32),
)


def _tc_scale_body(h_ref, deg8_ref, dinv16_ref, hs_ref):
    sel = jnp.repeat(jnp.eye(8, dtype=jnp.float32), 16, axis=1)
    deg = deg8_ref[0] + deg8_ref[1]
    deg16 = jnp.dot(deg, sel, preferred_element_type=jnp.float32) + 1.0
    dinv16 = lax.rsqrt(deg16)
    dinv16_ref[...] = dinv16
    hs_ref[...] = h_ref[...] * dinv16[:NPK]


_tc_scale = pl.pallas_call(
    _tc_scale_body,
    out_shape=(
        jax.ShapeDtypeStruct((NPK_PAD, 8 * DH), jnp.float32),
        jax.ShapeDtypeStruct((NPK, 8 * DH), jnp.float32),
    ),
)


def _tc_b_body(hs1_ref, agg_ref, dinv16_ref, w2bd_ref, b1_ref, hs2_ref):
    dinv16 = dinv16_ref[:NPK]
    agg = agg_ref[0, :NPK] + agg_ref[1, :NPK]
    out1 = (agg + hs1_ref[...]) * dinv16 + b1_ref[...]
    r = jnp.maximum(out1, 0.0)
    h2 = jnp.dot(r, w2bd_ref[...], preferred_element_type=jnp.float32)
    hs2_ref[...] = h2 * dinv16


_tc_b = pl.pallas_call(
    _tc_b_body,
    out_shape=jax.ShapeDtypeStruct((NPK, 8 * NCLS), jnp.float32),
)


def _tc_c_body(hs2_ref, agg_ref, dinv16_ref, b2_ref, out_ref):
    agg = agg_ref[0, :NPK] + agg_ref[1, :NPK]
    out_ref[...] = (agg + hs2_ref[...]) * dinv16_ref[:NPK] + b2_ref[...]


_tc_c = pl.pallas_call(
    _tc_c_body,
    out_shape=jax.ShapeDtypeStruct((NPK, 8 * NCLS), jnp.float32),
)


def _block_diag8(w):
    a, b = w.shape
    z = jnp.zeros((a, b), w.dtype)
    rows = [jnp.concatenate([w if i == j else z for j in range(8)], axis=1)
            for i in range(8)]
    return jnp.concatenate(rows, axis=0)


def kernel(x, edge_index, W1, b1, W2, b2):
    ei = edge_index.astype(jnp.int32).reshape(2, NCHUNKS, BATCH)
    src2d = ei[0]
    dst2d = ei[1]

    w1bd = _block_diag8(W1)
    w2bd = _block_diag8(W2)
    b1t = jnp.tile(b1, 8).reshape(1, 8 * DH)
    b2t = jnp.tile(b2, 8).reshape(1, 8 * NCLS)
    xr = x.reshape(NPK, 8 * DF)

    h1p = _tc_mm(xr, w1bd)
    degp = _deg_kernel(src2d, dst2d)
    deg8 = degp.reshape(NCORE, NPK_PAD, 8)
    dinv16, hs1p = _tc_scale(h1p, deg8)
    hs1 = hs1p.reshape(N, DH)
    agg1 = _agg_kernel(hs1, src2d, dst2d)
    agg1p = agg1.reshape(NCORE, NPK_PAD, 8 * DH)
    hs2p = _tc_b(hs1p, agg1p, dinv16, w2bd, b1t)
    hs2 = hs2p.reshape(N, DH)
    agg2 = _agg_kernel(hs2, src2d, dst2d)
    agg2p = agg2.reshape(NCORE, NPK_PAD, 8 * NCLS)
    outp = _tc_c(hs2p, agg2p, dinv16, b2t)
    return outp.reshape(N, NCLS)


# speedup vs baseline: 90.6380x; 1.1081x over previous
import functools

import jax
import jax.numpy as jnp
from jax import lax
from jax.experimental import pallas as pl
from jax.experimental.pallas import tpu as pltpu
from jax.experimental.pallas import tpu_sc as plsc

N = 10000
E = 320000
DF = 128
DH = 16
NCLS = 16

NCORE = 2
NSUB = 16
NTILE = NCORE * NSUB

N_PAD = 10240
ROWS_PER_TILE = N_PAD // NSUB
BATCH = 128
NCHUNKS = E // BATCH
TPC = 80
LAST_C = NCHUNKS - (NTILE - 1) * TPC

NPK = N // 8
NPK_PAD = N_PAD // 8

_mesh = plsc.VectorSubcoreMesh(core_axis_name="c", subcore_axis_name="s")
_sc_params = pltpu.CompilerParams(use_tc_tiling_on_sc=False)


@functools.partial(
    pl.kernel,
    out_type=jax.ShapeDtypeStruct((NCORE, N_PAD), jnp.float32),
    mesh=_mesh,
    scratch_types=[
        pltpu.VMEM_SHARED((N_PAD,), jnp.float32),
        pltpu.VMEM((TPC, BATCH), jnp.int32),
        pltpu.VMEM((BATCH,), jnp.float32),
        pltpu.VMEM((ROWS_PER_TILE,), jnp.float32),
        pltpu.SemaphoreType.DMA,
        pltpu.SemaphoreType.DMA,
        pltpu.SemaphoreType.DMA,
        pltpu.SemaphoreType.DMA,
    ],
    compiler_params=_sc_params,
)
def _deg_kernel(src_hbm, dst_hbm, out_hbm, acc, didx, ones_v, stage,
                s0, s1, s2, s3):
    del src_hbm
    cid = lax.axis_index("c")
    sid = lax.axis_index("s")
    tile = cid * NSUB + sid
    is_last = tile == NTILE - 1

    def _fill(i, _):
        ones_v[pl.ds(i * 16, 16)] = jnp.ones((16,), jnp.float32)
        return 0

    lax.fori_loop(0, BATCH // 16, _fill, 0, unroll=True)

    def _zero(i, _):
        stage[pl.ds(i * 16, 16)] = jnp.zeros((16,), jnp.float32)
        return 0

    lax.fori_loop(0, ROWS_PER_TILE // 16, _zero, 0)
    pltpu.sync_copy(stage, acc.at[pl.ds(sid * ROWS_PER_TILE, ROWS_PER_TILE)])

    @pl.when(jnp.logical_not(is_last))
    def _():
        pltpu.sync_copy(dst_hbm.at[pl.ds(tile * TPC, TPC)], didx)

    @pl.when(is_last)
    def _():
        pltpu.sync_copy(dst_hbm.at[pl.ds(tile * TPC, LAST_C)],
                        didx.at[pl.ds(0, LAST_C)])

    plsc.subcore_barrier()
    nquad = jnp.where(is_last, LAST_C // 4, TPC // 4)
    sems = (s0, s1, s2, s3)

    for b in range(4):
        pltpu.async_copy(ones_v, acc.at[didx.at[b]], sems[b], add=True)

    def _body(k, _):
        j = 4 * k
        for b in range(4):
            pltpu.make_async_copy(ones_v, acc.at[didx.at[j + b]],
                                  sems[b]).wait()
            pltpu.async_copy(ones_v, acc.at[didx.at[j + 4 + b]], sems[b],
                             add=True)
        return 0

    lax.fori_loop(0, nquad - 1, _body, 0)
    jT = 4 * (nquad - 1)
    for b in range(4):
        pltpu.make_async_copy(ones_v, acc.at[didx.at[jT + b]], sems[b]).wait()
    plsc.subcore_barrier()
    sl = pl.ds(sid * ROWS_PER_TILE, ROWS_PER_TILE)
    pltpu.sync_copy(acc.at[sl], out_hbm.at[cid, sl])


@functools.partial(
    pl.kernel,
    out_type=jax.ShapeDtypeStruct((NCORE, N_PAD, DH), jnp.float32),
    mesh=_mesh,
    scratch_types=[
        pltpu.VMEM_SHARED((N_PAD, DH), jnp.float32),
        pltpu.VMEM((TPC, BATCH), jnp.int32),
        pltpu.VMEM((TPC, BATCH), jnp.int32),
        *([pltpu.VMEM((BATCH, DH), jnp.float32)] * 8),
        pltpu.VMEM((ROWS_PER_TILE, DH), jnp.float32),
        *([pltpu.SemaphoreType.DMA] * 16),
    ],
    compiler_params=_sc_params,
)
def _agg_kernel(hs_hbm, src_hbm, dst_hbm, out_hbm,
                acc, sidx, didx, *bufs_and_sems):
    rows = bufs_and_sems[:8]
    stage = bufs_and_sems[8]
    gsem = bufs_and_sems[9:17]
    ssem = bufs_and_sems[17:25]
    cid = lax.axis_index("c")
    sid = lax.axis_index("s")
    tile = cid * NSUB + sid
    is_last = tile == NTILE - 1

    def _zero(i, _):
        stage[i, :] = jnp.zeros((DH,), jnp.float32)
        return 0

    lax.fori_loop(0, ROWS_PER_TILE, _zero, 0, unroll=8)
    pltpu.sync_copy(stage, acc.at[pl.ds(sid * ROWS_PER_TILE, ROWS_PER_TILE)])

    @pl.when(jnp.logical_not(is_last))
    def _():
        pltpu.sync_copy(src_hbm.at[pl.ds(tile * TPC, TPC)], sidx)
        pltpu.sync_copy(dst_hbm.at[pl.ds(tile * TPC, TPC)], didx)

    @pl.when(is_last)
    def _():
        pltpu.sync_copy(src_hbm.at[pl.ds(tile * TPC, LAST_C)],
                        sidx.at[pl.ds(0, LAST_C)])
        pltpu.sync_copy(dst_hbm.at[pl.ds(tile * TPC, LAST_C)],
                        didx.at[pl.ds(0, LAST_C)])

    plsc.subcore_barrier()

    def _run_pipe(nch, depth):
        for b in range(depth):
            pltpu.async_copy(hs_hbm.at[sidx.at[b]], rows[b], gsem[b])

        def _bd(k, _):
            j = depth * k
            for b in range(depth):
                pltpu.make_async_copy(
                    hs_hbm.at[sidx.at[j + b]], rows[b], gsem[b]).wait()
                pltpu.async_copy(rows[b], acc.at[didx.at[j + b]], ssem[b],
                                 add=True)
            for b in range(depth):
                pltpu.make_async_copy(rows[b], acc.at[didx.at[j + b]],
                                      ssem[b]).wait()
                pltpu.async_copy(hs_hbm.at[sidx.at[j + depth + b]], rows[b],
                                 gsem[b])
            return 0

        lax.fori_loop(0, nch // depth - 1, _bd, 0)
        jT = nch - depth
        for b in range(depth):
            pltpu.make_async_copy(hs_hbm.at[sidx.at[jT + b]], rows[b],
                                  gsem[b]).wait()
            pltpu.sync_copy(rows[b], acc.at[didx.at[jT + b]], add=True)

    @pl.when(jnp.logical_not(is_last))
    def _():
        _run_pipe(TPC, 8)

    @pl.when(is_last)
    def _():
        _run_pipe(LAST_C, 4)

    plsc.subcore_barrier()
    sl = pl.ds(sid * ROWS_PER_TILE, ROWS_PER_TILE)
    pltpu.sync_copy(acc.at[sl], out_hbm.at[cid, sl])


def _tc_mm_body(xr_ref, w1bd_ref, h_ref):
    h_ref[...] = jnp.dot(xr_ref[...], w1bd_ref[...],
                         preferred_element_type=jnp.float32)


_tc_mm = pl.pallas_call(
    _tc_mm_body,
    out_shape=jax.ShapeDtypeStruct((NPK, 8 * DH), jnp.float32),
)


def _tc_scale_body(h_ref, deg8_ref, dinv16_ref, hs_ref):
    sel = jnp.repeat(jnp.eye(8, dtype=jnp.float32), 16, axis=1)
    deg = deg8_ref[0] + deg8_ref[1]
    deg16 = jnp.dot(deg, sel, preferred_element_type=jnp.float32) + 1.0
    dinv16 = lax.rsqrt(deg16)
    dinv16_ref[...] = dinv16
    hs_ref[...] = h_ref[...] * dinv16[:NPK]


_tc_scale = pl.pallas_call(
    _tc_scale_body,
    out_shape=(
        jax.ShapeDtypeStruct((NPK_PAD, 8 * DH), jnp.float32),
        jax.ShapeDtypeStruct((NPK, 8 * DH), jnp.float32),
    ),
)


def _tc_b_body(hs1_ref, agg_ref, dinv16_ref, w2bd_ref, b1_ref, hs2_ref):
    dinv16 = dinv16_ref[:NPK]
    agg = agg_ref[0, :NPK] + agg_ref[1, :NPK]
    out1 = (agg + hs1_ref[...]) * dinv16 + b1_ref[...]
    r = jnp.maximum(out1, 0.0)
    h2 = jnp.dot(r, w2bd_ref[...], preferred_element_type=jnp.float32)
    hs2_ref[...] = h2 * dinv16


_tc_b = pl.pallas_call(
    _tc_b_body,
    out_shape=jax.ShapeDtypeStruct((NPK, 8 * NCLS), jnp.float32),
)


def _tc_c_body(hs2_ref, agg_ref, dinv16_ref, b2_ref, out_ref):
    agg = agg_ref[0, :NPK] + agg_ref[1, :NPK]
    out_ref[...] = (agg + hs2_ref[...]) * dinv16_ref[:NPK] + b2_ref[...]


_tc_c = pl.pallas_call(
    _tc_c_body,
    out_shape=jax.ShapeDtypeStruct((NPK, 8 * NCLS), jnp.float32),
)


def _block_diag8(w):
    a, b = w.shape
    z = jnp.zeros((a, b), w.dtype)
    rows = [jnp.concatenate([w if i == j else z for j in range(8)], axis=1)
            for i in range(8)]
    return jnp.concatenate(rows, axis=0)


def kernel(x, edge_index, W1, b1, W2, b2):
    ei = edge_index.astype(jnp.int32).reshape(2, NCHUNKS, BATCH)
    src2d = ei[0]
    dst2d = ei[1]

    w1bd = _block_diag8(W1)
    w2bd = _block_diag8(W2)
    b1t = jnp.tile(b1, 8).reshape(1, 8 * DH)
    b2t = jnp.tile(b2, 8).reshape(1, 8 * NCLS)
    xr = x.reshape(NPK, 8 * DF)

    h1p = _tc_mm(xr, w1bd)
    degp = _deg_kernel(src2d, dst2d)
    deg8 = degp.reshape(NCORE, NPK_PAD, 8)
    dinv16, hs1p = _tc_scale(h1p, deg8)
    hs1 = hs1p.reshape(N, DH)
    agg1 = _agg_kernel(hs1, src2d, dst2d)
    agg1p = agg1.reshape(NCORE, NPK_PAD, 8 * DH)
    hs2p = _tc_b(hs1p, agg1p, dinv16, w2bd, b1t)
    hs2 = hs2p.reshape(N, DH)
    agg2 = _agg_kernel(hs2, src2d, dst2d)
    agg2p = agg2.reshape(NCORE, NPK_PAD, 8 * NCLS)
    outp = _tc_c(hs2p, agg2p, dinv16, b2t)
    return outp.reshape(N, NCLS)
